# baseline (speedup 1.0000x reference)
import jax
import jax.numpy as jnp
from jax.experimental import pallas as pl

_BLOCK_ROWS = 512


def _copy_kernel(in_ref, out_ref):
    out_ref[...] = in_ref[...]


def kernel(attention_mask, inputs_embeds):
    del attention_mask
    b, s, h = inputs_embeds.shape
    rows = b * s
    x = inputs_embeds.reshape(rows, h)
    out = pl.pallas_call(
        _copy_kernel,
        out_shape=jax.ShapeDtypeStruct((rows, h), x.dtype),
        grid=(rows // _BLOCK_ROWS,),
        in_specs=[pl.BlockSpec((_BLOCK_ROWS, h), lambda i: (i, 0))],
        out_specs=pl.BlockSpec((_BLOCK_ROWS, h), lambda i: (i, 0)),
    )(x)
    return out.reshape(b, s, h)
